# baseline (device time: 94551 ns/iter reference)
import jax
import jax.numpy as jnp
from jax import lax
from jax.experimental import pallas as pl
from jax.experimental.pallas import tpu as pltpu

N_DEV = 32
M_BLK = 256
WINDOW = 16


def kernel(x, w_mat):
    m_all, k_shard = x.shape
    k_all, n = w_mat.shape

    def body(idx_ref, x_ref, w_ref, out_ref, xbf_ref, comm_ref,
             send_sems, recv_sems):
        s = pl.program_id(0)
        my_i = lax.axis_index("i")

        def start_round(r):
            t = lax.rem(my_i + r, N_DEV)
            pltpu.make_async_remote_copy(
                src_ref=xbf_ref.at[pl.ds(t * M_BLK, M_BLK)],
                dst_ref=comm_ref.at[r],
                send_sem=send_sems.at[r],
                recv_sem=recv_sems.at[r],
                device_id=(t,),
                device_id_type=pl.DeviceIdType.MESH,
            ).start()

        @pl.when(s == 0)
        def _():
            xbf_ref[...] = x_ref[...].astype(jnp.bfloat16)

            barrier_sem = pltpu.get_barrier_semaphore()
            for r in range(1, N_DEV):
                t = lax.rem(my_i + r, N_DEV)
                pl.semaphore_signal(
                    barrier_sem, inc=1,
                    device_id=(t,), device_id_type=pl.DeviceIdType.MESH,
                )
            pl.semaphore_wait(barrier_sem, N_DEV - 1)

            comm_ref[0] = xbf_ref[pl.ds(my_i * M_BLK, M_BLK), :]

            for r in range(1, min(WINDOW + 1, N_DEV)):
                start_round(r)

        for r in range(WINDOW + 1, N_DEV):

            @pl.when(s == r - WINDOW)
            def _(r=r):
                start_round(r)

        @pl.when(s > 0)
        def _():
            pltpu.make_async_remote_copy(
                src_ref=xbf_ref.at[pl.ds(0, M_BLK)],
                dst_ref=comm_ref.at[0],
                send_sem=send_sems.at[0],
                recv_sem=recv_sems.at[s],
                device_id=(my_i,),
                device_id_type=pl.DeviceIdType.MESH,
            ).wait_recv()

        @pl.when(s == N_DEV - 1)
        def _():
            for r in range(1, N_DEV):
                pltpu.make_async_remote_copy(
                    src_ref=xbf_ref.at[pl.ds(0, M_BLK)],
                    dst_ref=comm_ref.at[0],
                    send_sem=send_sems.at[r],
                    recv_sem=recv_sems.at[0],
                    device_id=(my_i,),
                    device_id_type=pl.DeviceIdType.MESH,
                ).wait_send()

        contrib = jnp.dot(
            comm_ref[s].astype(jnp.float32), w_ref[...],
            preferred_element_type=jnp.float32,
        )

        @pl.when(s == 0)
        def _():
            out_ref[...] = contrib

        @pl.when(jnp.logical_and(s > 0, s < N_DEV - 1))
        def _():
            out_ref[...] += contrib

        @pl.when(s == N_DEV - 1)
        def _():
            out_ref[...] = jnp.maximum(out_ref[...] + contrib, 0.0)

    my_i = lax.axis_index("i")
    perm = lax.rem(
        my_i - jnp.arange(N_DEV, dtype=jnp.int32) + N_DEV, N_DEV
    ).astype(jnp.int32)

    grid_spec = pltpu.PrefetchScalarGridSpec(
        num_scalar_prefetch=1,
        grid=(N_DEV,),
        in_specs=[
            pl.BlockSpec((m_all, k_shard), lambda s, idx: (0, 0)),
            pl.BlockSpec((M_BLK, n), lambda s, idx: (idx[s], 0)),
        ],
        out_specs=pl.BlockSpec((M_BLK, n), lambda s, idx: (0, 0)),
        scratch_shapes=[
            pltpu.VMEM((m_all, k_shard), jnp.bfloat16),
            pltpu.VMEM((N_DEV, M_BLK, M_BLK), jnp.bfloat16),
            pltpu.SemaphoreType.DMA((N_DEV,)),
            pltpu.SemaphoreType.DMA((N_DEV,)),
        ],
    )

    return pl.pallas_call(
        body,
        grid_spec=grid_spec,
        out_shape=jax.ShapeDtypeStruct((M_BLK, n), jnp.float32),
        compiler_params=pltpu.CompilerParams(
            dimension_semantics=("arbitrary",),
            collective_id=0,
        ),
    )(perm, x, w_mat)


# device time: 93523 ns/iter; 1.0110x vs baseline; 1.0110x over previous
import jax
import jax.numpy as jnp
from jax import lax
from jax.experimental import pallas as pl
from jax.experimental.pallas import tpu as pltpu

N_DEV = 32
M_BLK = 256
WINDOW = 4


def kernel(x, w_mat):
    m_all, k_shard = x.shape
    k_all, n = w_mat.shape

    def body(idx_ref, x_ref, w_ref, out_ref, comm_ref,
             send_sems, recv_sems):
        s = pl.program_id(0)
        my_i = lax.axis_index("i")

        def start_round(r):
            t = lax.rem(my_i + r, N_DEV)
            pltpu.make_async_remote_copy(
                src_ref=x_ref.at[pl.ds(t * M_BLK, M_BLK)],
                dst_ref=comm_ref.at[r],
                send_sem=send_sems.at[r],
                recv_sem=recv_sems.at[r],
                device_id=(t,),
                device_id_type=pl.DeviceIdType.MESH,
            ).start()

        @pl.when(s == 0)
        def _():
            barrier_sem = pltpu.get_barrier_semaphore()
            for r in range(1, N_DEV):
                t = lax.rem(my_i + r, N_DEV)
                pl.semaphore_signal(
                    barrier_sem, inc=1,
                    device_id=(t,), device_id_type=pl.DeviceIdType.MESH,
                )
            pl.semaphore_wait(barrier_sem, N_DEV - 1)

            comm_ref[0] = x_ref[pl.ds(my_i * M_BLK, M_BLK), :]

            for r in range(1, min(WINDOW + 1, N_DEV)):
                start_round(r)

        for r in range(WINDOW + 1, N_DEV):

            @pl.when(s == r - WINDOW)
            def _(r=r):
                start_round(r)

        @pl.when(s > 0)
        def _():
            pltpu.make_async_remote_copy(
                src_ref=x_ref.at[pl.ds(0, M_BLK)],
                dst_ref=comm_ref.at[0],
                send_sem=send_sems.at[0],
                recv_sem=recv_sems.at[s],
                device_id=(my_i,),
                device_id_type=pl.DeviceIdType.MESH,
            ).wait_recv()

        @pl.when(s == N_DEV - 1)
        def _():
            for r in range(1, N_DEV):
                pltpu.make_async_remote_copy(
                    src_ref=x_ref.at[pl.ds(0, M_BLK)],
                    dst_ref=comm_ref.at[0],
                    send_sem=send_sems.at[r],
                    recv_sem=recv_sems.at[0],
                    device_id=(my_i,),
                    device_id_type=pl.DeviceIdType.MESH,
                ).wait_send()

        contrib = jnp.dot(
            comm_ref[s].astype(jnp.float32), w_ref[...],
            preferred_element_type=jnp.float32,
        )

        @pl.when(s == 0)
        def _():
            out_ref[...] = contrib

        @pl.when(jnp.logical_and(s > 0, s < N_DEV - 1))
        def _():
            out_ref[...] += contrib

        @pl.when(s == N_DEV - 1)
        def _():
            out_ref[...] = jnp.maximum(out_ref[...] + contrib, 0.0)

    my_i = lax.axis_index("i")
    perm = lax.rem(
        my_i - jnp.arange(N_DEV, dtype=jnp.int32) + N_DEV, N_DEV
    ).astype(jnp.int32)

    grid_spec = pltpu.PrefetchScalarGridSpec(
        num_scalar_prefetch=1,
        grid=(N_DEV,),
        in_specs=[
            pl.BlockSpec((m_all, k_shard), lambda s, idx: (0, 0)),
            pl.BlockSpec((M_BLK, n), lambda s, idx: (idx[s], 0)),
        ],
        out_specs=pl.BlockSpec((M_BLK, n), lambda s, idx: (0, 0)),
        scratch_shapes=[
            pltpu.VMEM((N_DEV, M_BLK, M_BLK), jnp.bfloat16),
            pltpu.SemaphoreType.DMA((N_DEV,)),
            pltpu.SemaphoreType.DMA((N_DEV,)),
        ],
    )

    return pl.pallas_call(
        body,
        grid_spec=grid_spec,
        out_shape=jax.ShapeDtypeStruct((M_BLK, n), jnp.float32),
        compiler_params=pltpu.CompilerParams(
            dimension_semantics=("arbitrary",),
            collective_id=0,
        ),
    )(perm, x.astype(jnp.bfloat16), w_mat)


# device time: 91797 ns/iter; 1.0300x vs baseline; 1.0188x over previous
import jax
import jax.numpy as jnp
from jax import lax
from jax.experimental import pallas as pl
from jax.experimental.pallas import tpu as pltpu

N_DEV = 32
M_BLK = 256
WINDOW = 8


def kernel(x, w_mat):
    m_all, k_shard = x.shape
    k_all, n = w_mat.shape

    def body(idx_ref, x_ref, w_ref, out_ref, xbf_ref, comm_ref,
             send_sems, recv_sems):
        s = pl.program_id(0)
        my_i = lax.axis_index("i")

        def start_round(r):
            t = lax.rem(my_i + r, N_DEV)
            pltpu.make_async_remote_copy(
                src_ref=xbf_ref.at[pl.ds(t * M_BLK, M_BLK)],
                dst_ref=comm_ref.at[r],
                send_sem=send_sems.at[r],
                recv_sem=recv_sems.at[r],
                device_id=(t,),
                device_id_type=pl.DeviceIdType.MESH,
            ).start()

        @pl.when(s == 0)
        def _():
            xbf_ref[...] = x_ref[...].astype(jnp.bfloat16)

            barrier_sem = pltpu.get_barrier_semaphore()
            for r in range(1, N_DEV):
                t = lax.rem(my_i + r, N_DEV)
                pl.semaphore_signal(
                    barrier_sem, inc=1,
                    device_id=(t,), device_id_type=pl.DeviceIdType.MESH,
                )
            pl.semaphore_wait(barrier_sem, N_DEV - 1)

            comm_ref[0] = xbf_ref[pl.ds(my_i * M_BLK, M_BLK), :]

            for r in range(1, min(WINDOW + 1, N_DEV)):
                start_round(r)

        for r in range(WINDOW + 1, N_DEV):

            @pl.when(s == r - WINDOW)
            def _(r=r):
                start_round(r)

        @pl.when(s > 0)
        def _():
            pltpu.make_async_remote_copy(
                src_ref=xbf_ref.at[pl.ds(0, M_BLK)],
                dst_ref=comm_ref.at[0],
                send_sem=send_sems.at[0],
                recv_sem=recv_sems.at[s],
                device_id=(my_i,),
                device_id_type=pl.DeviceIdType.MESH,
            ).wait_recv()

        @pl.when(s == N_DEV - 1)
        def _():
            for r in range(1, N_DEV):
                pltpu.make_async_remote_copy(
                    src_ref=xbf_ref.at[pl.ds(0, M_BLK)],
                    dst_ref=comm_ref.at[0],
                    send_sem=send_sems.at[r],
                    recv_sem=recv_sems.at[0],
                    device_id=(my_i,),
                    device_id_type=pl.DeviceIdType.MESH,
                ).wait_send()

        contrib = jnp.dot(
            comm_ref[s].astype(jnp.float32), w_ref[...],
            preferred_element_type=jnp.float32,
        )

        @pl.when(s == 0)
        def _():
            out_ref[...] = contrib

        @pl.when(jnp.logical_and(s > 0, s < N_DEV - 1))
        def _():
            out_ref[...] += contrib

        @pl.when(s == N_DEV - 1)
        def _():
            out_ref[...] = jnp.maximum(out_ref[...] + contrib, 0.0)

    my_i = lax.axis_index("i")
    perm = lax.rem(
        my_i - jnp.arange(N_DEV, dtype=jnp.int32) + N_DEV, N_DEV
    ).astype(jnp.int32)

    grid_spec = pltpu.PrefetchScalarGridSpec(
        num_scalar_prefetch=1,
        grid=(N_DEV,),
        in_specs=[
            pl.BlockSpec((m_all, k_shard), lambda s, idx: (0, 0)),
            pl.BlockSpec((M_BLK, n), lambda s, idx: (idx[s], 0)),
        ],
        out_specs=pl.BlockSpec((M_BLK, n), lambda s, idx: (0, 0)),
        scratch_shapes=[
            pltpu.VMEM((m_all, k_shard), jnp.bfloat16),
            pltpu.VMEM((N_DEV, M_BLK, M_BLK), jnp.bfloat16),
            pltpu.SemaphoreType.DMA((N_DEV,)),
            pltpu.SemaphoreType.DMA((N_DEV,)),
        ],
    )

    return pl.pallas_call(
        body,
        grid_spec=grid_spec,
        out_shape=jax.ShapeDtypeStruct((M_BLK, n), jnp.float32),
        compiler_params=pltpu.CompilerParams(
            dimension_semantics=("arbitrary",),
            collective_id=0,
        ),
    )(perm, x, w_mat)


# device time: 90534 ns/iter; 1.0444x vs baseline; 1.0140x over previous
import jax
import jax.numpy as jnp
from jax import lax
from jax.experimental import pallas as pl
from jax.experimental.pallas import tpu as pltpu

N_DEV = 32
M_BLK = 256
WINDOW = 8
W_DEPTH = 6


def kernel(x, w_mat):
    m_all, k_shard = x.shape
    k_all, n = w_mat.shape

    def body(idx_ref, x_ref, w_ref, out_ref, xbf_ref, comm_ref, wbuf_ref,
             send_sems, recv_sems, wdma_sems):
        s = pl.program_id(0)
        my_i = lax.axis_index("i")

        def fetch_w(q, slot):
            pltpu.make_async_copy(
                w_ref.at[pl.ds(idx_ref[q] * M_BLK, M_BLK)],
                wbuf_ref.at[slot],
                wdma_sems.at[slot],
            ).start()

        def start_round(r):
            t = lax.rem(my_i + r, N_DEV)
            pltpu.make_async_remote_copy(
                src_ref=xbf_ref.at[pl.ds(t * M_BLK, M_BLK)],
                dst_ref=comm_ref.at[r],
                send_sem=send_sems.at[r],
                recv_sem=recv_sems.at[r],
                device_id=(t,),
                device_id_type=pl.DeviceIdType.MESH,
            ).start()

        @pl.when(s == 0)
        def _():
            for q in range(W_DEPTH):
                fetch_w(q, q)

            xbf_ref[...] = x_ref[...].astype(jnp.bfloat16)

            barrier_sem = pltpu.get_barrier_semaphore()
            for r in range(1, N_DEV):
                t = lax.rem(my_i + r, N_DEV)
                pl.semaphore_signal(
                    barrier_sem, inc=1,
                    device_id=(t,), device_id_type=pl.DeviceIdType.MESH,
                )
            pl.semaphore_wait(barrier_sem, N_DEV - 1)

            comm_ref[0] = xbf_ref[pl.ds(my_i * M_BLK, M_BLK), :]

            for r in range(1, min(WINDOW + 1, N_DEV)):
                start_round(r)

        for r in range(WINDOW + 1, N_DEV):

            @pl.when(s == r - WINDOW)
            def _(r=r):
                start_round(r)

        @pl.when(s > 0)
        def _():
            pltpu.make_async_remote_copy(
                src_ref=xbf_ref.at[pl.ds(0, M_BLK)],
                dst_ref=comm_ref.at[0],
                send_sem=send_sems.at[0],
                recv_sem=recv_sems.at[s],
                device_id=(my_i,),
                device_id_type=pl.DeviceIdType.MESH,
            ).wait_recv()

        @pl.when(s == N_DEV - 1)
        def _():
            for r in range(1, N_DEV):
                pltpu.make_async_remote_copy(
                    src_ref=xbf_ref.at[pl.ds(0, M_BLK)],
                    dst_ref=comm_ref.at[0],
                    send_sem=send_sems.at[r],
                    recv_sem=recv_sems.at[0],
                    device_id=(my_i,),
                    device_id_type=pl.DeviceIdType.MESH,
                ).wait_send()

        slot = lax.rem(s, W_DEPTH)
        pltpu.make_async_copy(
            w_ref.at[pl.ds(0, M_BLK)],
            wbuf_ref.at[0],
            wdma_sems.at[slot],
        ).wait()
        contrib = jnp.dot(
            comm_ref[s].astype(jnp.float32), wbuf_ref[slot],
            preferred_element_type=jnp.float32,
        )

        @pl.when(s + W_DEPTH < N_DEV)
        def _():
            fetch_w(s + W_DEPTH, slot)

        @pl.when(s == 0)
        def _():
            out_ref[...] = contrib

        @pl.when(jnp.logical_and(s > 0, s < N_DEV - 1))
        def _():
            out_ref[...] += contrib

        @pl.when(s == N_DEV - 1)
        def _():
            out_ref[...] = jnp.maximum(out_ref[...] + contrib, 0.0)

    my_i = lax.axis_index("i")
    perm = lax.rem(
        my_i - jnp.arange(N_DEV, dtype=jnp.int32) + N_DEV, N_DEV
    ).astype(jnp.int32)

    grid_spec = pltpu.PrefetchScalarGridSpec(
        num_scalar_prefetch=1,
        grid=(N_DEV,),
        in_specs=[
            pl.BlockSpec((m_all, k_shard), lambda s, idx: (0, 0)),
            pl.BlockSpec(memory_space=pl.ANY),
        ],
        out_specs=pl.BlockSpec((M_BLK, n), lambda s, idx: (0, 0)),
        scratch_shapes=[
            pltpu.VMEM((m_all, k_shard), jnp.bfloat16),
            pltpu.VMEM((N_DEV, M_BLK, M_BLK), jnp.bfloat16),
            pltpu.VMEM((W_DEPTH, M_BLK, n), jnp.float32),
            pltpu.SemaphoreType.DMA((N_DEV,)),
            pltpu.SemaphoreType.DMA((N_DEV,)),
            pltpu.SemaphoreType.DMA((W_DEPTH,)),
        ],
    )

    return pl.pallas_call(
        body,
        grid_spec=grid_spec,
        out_shape=jax.ShapeDtypeStruct((M_BLK, n), jnp.float32),
        compiler_params=pltpu.CompilerParams(
            dimension_semantics=("arbitrary",),
            collective_id=0,
            vmem_limit_bytes=56 * 1024 * 1024,
        ),
    )(perm, x, w_mat)
